# baseline (speedup 1.0000x reference)
import functools

import jax
import jax.numpy as jnp
from jax.experimental import pallas as pl

SCALE_LIST = (1, 2, 4, 8, 16, 32, 64)


def _downsample(x, s, seq):
    f = seq // s
    x3 = x.reshape(s, f, x.shape[1])
    return 0.5 * (x3[:, f // 2 - 1, :] + x3[:, f // 2, :])


def _upsample(q, s, seq, n):
    g = seq // s
    u = jax.lax.broadcasted_iota(jnp.int32, (seq, 1), 0)
    pos = jnp.clip((u.astype(jnp.float32) + 0.5) * (s / seq) - 0.5,
                   0.0, float(s - 1))
    lo = jnp.floor(pos)
    w = pos - lo
    base = (u // g).astype(jnp.float32)
    lo_eq = lo == base
    if s == 1:
        return jnp.broadcast_to(q, (seq, n))
    cc = jnp.broadcast_to(q[:, None, :], (s, g, n)).reshape(seq, n)
    qn = jnp.concatenate([q[1:], q[-1:]], axis=0)
    qp = jnp.concatenate([q[:1], q[:-1]], axis=0)
    dd = jnp.broadcast_to(qn[:, None, :], (s, g, n)).reshape(seq, n)
    ee = jnp.broadcast_to(qp[:, None, :], (s, g, n)).reshape(seq, n)
    low = jnp.where(lo_eq, cc, ee)
    high = jnp.where(lo_eq, dd, cc)
    return low * (1.0 - w) + high * w


def _vq_body(z_ref, c_ref, ct_ref, f_ref, loss_ref, *, scales, bb, seq, dim, k):
    x0 = z_ref[...]
    cb = c_ref[...]
    cbt = ct_ref[...]
    n = bb * dim
    c2 = jnp.sum(cbt * cbt, axis=0, keepdims=True)

    f = jnp.zeros_like(x0)
    for s in scales:
        xr = x0 - f
        rsm = xr if s == seq else _downsample(xr, s, seq)
        r = jnp.concatenate(
            [rsm[:, b * dim:(b + 1) * dim] for b in range(bb)], axis=0)
        r2 = jnp.sum(r * r, axis=1, keepdims=True)
        e = jnp.dot(r.astype(jnp.bfloat16), cbt.astype(jnp.bfloat16),
                    preferred_element_type=jnp.float32)
        dist = (r2 - 2.0 * e) + c2
        m = jnp.min(dist, axis=1, keepdims=True)
        kiota = jax.lax.broadcasted_iota(jnp.int32, dist.shape, 1)
        idx = jnp.min(jnp.where(dist == m, kiota, k), axis=1, keepdims=True)
        onehot = (kiota == idx).astype(jnp.float32)
        q = jnp.dot(onehot, cb, preferred_element_type=jnp.float32,
                    precision=jax.lax.Precision.HIGHEST)
        qsm = jnp.concatenate(
            [q[b * s:(b + 1) * s, :] for b in range(bb)], axis=1)
        f = f + (qsm if s == seq else _upsample(qsm, s, seq, n))

    f_ref[...] = f
    part = jnp.sum((f - x0) ** 2)

    @pl.when(pl.program_id(0) == 0)
    def _init():
        loss_ref[...] = jnp.zeros_like(loss_ref)

    loss_ref[...] = loss_ref[...] + part


def kernel(z, codebook):
    b, seq, dim = z.shape
    k = codebook.shape[0]
    bb = 4
    grid = b // bb

    z2 = z.transpose(1, 0, 2).reshape(seq, b * dim)
    cbt = codebook.T

    f2, loss_sum = pl.pallas_call(
        functools.partial(_vq_body, scales=SCALE_LIST, bb=bb,
                          seq=seq, dim=dim, k=k),
        grid=(grid,),
        in_specs=[
            pl.BlockSpec((seq, bb * dim), lambda i: (0, i)),
            pl.BlockSpec((k, dim), lambda i: (0, 0)),
            pl.BlockSpec((dim, k), lambda i: (0, 0)),
        ],
        out_specs=[
            pl.BlockSpec((seq, bb * dim), lambda i: (0, i)),
            pl.BlockSpec((1, 1), lambda i: (0, 0)),
        ],
        out_shape=[
            jax.ShapeDtypeStruct((seq, b * dim), jnp.float32),
            jax.ShapeDtypeStruct((1, 1), jnp.float32),
        ],
    )(z2, codebook, cbt)

    f_hat = f2.reshape(seq, b, dim).transpose(1, 0, 2)
    loss = loss_sum[0, 0] / (b * seq * dim)
    return f_hat, loss, loss


# speedup vs baseline: 2.4402x; 2.4402x over previous
import functools

import jax
import jax.numpy as jnp
from jax.experimental import pallas as pl

SCALE_LIST = (1, 2, 4, 8, 16, 32, 64)
KCHUNK = 2048


def _downsample(x, s, seq):
    f = seq // s
    x3 = x.reshape(s, f, x.shape[1])
    return 0.5 * (x3[:, f // 2 - 1, :] + x3[:, f // 2, :])


def _upsample(q, s, seq, n):
    g = seq // s
    u = jax.lax.broadcasted_iota(jnp.int32, (seq, 1), 0)
    pos = jnp.clip((u.astype(jnp.float32) + 0.5) * (s / seq) - 0.5,
                   0.0, float(s - 1))
    lo = jnp.floor(pos)
    w = pos - lo
    base = (u // g).astype(jnp.float32)
    lo_eq = lo == base
    if s == 1:
        return jnp.broadcast_to(q, (seq, n))
    cc = jnp.broadcast_to(q[:, None, :], (s, g, n)).reshape(seq, n)
    qn = jnp.concatenate([q[1:], q[-1:]], axis=0)
    qp = jnp.concatenate([q[:1], q[:-1]], axis=0)
    dd = jnp.broadcast_to(qn[:, None, :], (s, g, n)).reshape(seq, n)
    ee = jnp.broadcast_to(qp[:, None, :], (s, g, n)).reshape(seq, n)
    low = jnp.where(lo_eq, cc, ee)
    high = jnp.where(lo_eq, dd, cc)
    return low * (1.0 - w) + high * w


def _vq_body(z_ref, ct_ref, cb_ref, f_ref, loss_ref, *,
             scales, bb, seq, dim, k):
    x0 = z_ref[...]
    n = bb * dim
    cbt = ct_ref[...]
    c2 = jnp.sum(cbt * cbt, axis=0, keepdims=True)

    kch = min(KCHUNK, k)
    nchunks = k // kch
    f = jnp.zeros_like(x0)
    for s in scales:
        xr = x0 - f
        rsm = xr if s == seq else _downsample(xr, s, seq)
        r = jnp.concatenate(
            [rsm[:, b * dim:(b + 1) * dim] for b in range(bb)], axis=0)
        rows = bb * s
        r2 = jnp.sum(r * r, axis=1, keepdims=True)
        best = jnp.full((rows, 1), jnp.inf, dtype=jnp.float32)
        bq = jnp.zeros((rows, dim), dtype=jnp.float32)
        kiota = jax.lax.broadcasted_iota(jnp.int32, (rows, kch), 1)
        for c in range(nchunks):
            lo_k = c * kch
            e = jnp.dot(r.astype(jnp.bfloat16),
                        cbt[:, lo_k:lo_k + kch].astype(jnp.bfloat16),
                        preferred_element_type=jnp.float32)
            dist = (r2 - 2.0 * e) + c2[:, lo_k:lo_k + kch]
            cmin = jnp.min(dist, axis=1, keepdims=True)
            cidx = jnp.min(jnp.where(dist == cmin, kiota, k),
                           axis=1, keepdims=True)
            onehot = (kiota == cidx).astype(jnp.float32)
            cq = jnp.dot(onehot, cb_ref[lo_k:lo_k + kch, :],
                         preferred_element_type=jnp.float32,
                         precision=jax.lax.Precision.HIGHEST)
            better = cmin < best
            best = jnp.where(better, cmin, best)
            bq = jnp.where(better, cq, bq)
        qsm = jnp.concatenate(
            [bq[b * s:(b + 1) * s, :] for b in range(bb)], axis=1)
        f = f + (qsm if s == seq else _upsample(qsm, s, seq, n))

    f_ref[...] = f
    part = jnp.sum((f - x0) ** 2)

    @pl.when(pl.program_id(0) == 0)
    def _init():
        loss_ref[...] = jnp.zeros_like(loss_ref)

    loss_ref[...] = loss_ref[...] + part


def kernel(z, codebook):
    b, seq, dim = z.shape
    k = codebook.shape[0]
    bb = 16
    grid = b // bb

    z2 = z.transpose(1, 0, 2).reshape(seq, b * dim)
    cbt = codebook.T

    f2, loss_sum = pl.pallas_call(
        functools.partial(_vq_body, scales=SCALE_LIST, bb=bb,
                          seq=seq, dim=dim, k=k),
        grid=(grid,),
        in_specs=[
            pl.BlockSpec((seq, bb * dim), lambda i: (0, i)),
            pl.BlockSpec((dim, k), lambda i: (0, 0)),
            pl.BlockSpec((k, dim), lambda i: (0, 0)),
        ],
        out_specs=[
            pl.BlockSpec((seq, bb * dim), lambda i: (0, i)),
            pl.BlockSpec((1, 1), lambda i: (0, 0)),
        ],
        out_shape=[
            jax.ShapeDtypeStruct((seq, b * dim), jnp.float32),
            jax.ShapeDtypeStruct((1, 1), jnp.float32),
        ],
    )(z2, cbt, codebook)

    f_hat = f2.reshape(seq, b, dim).transpose(1, 0, 2)
    loss = loss_sum[0, 0] / (b * seq * dim)
    return f_hat, loss, loss
